# baseline (speedup 1.0000x reference)
import jax
import jax.numpy as jnp
from jax.experimental import pallas as pl
from jax.experimental.pallas import tpu as pltpu


def _mlp_kernel(eu_ref, ev_ref, w1at_ref, w1bt_ref, b1_ref, w2_ref, out_ref):
    h = (jnp.dot(eu_ref[...], w1at_ref[...], preferred_element_type=jnp.float32)
         + jnp.dot(ev_ref[...], w1bt_ref[...], preferred_element_type=jnp.float32))
    h = jnp.maximum(h + b1_ref[...], 0.0)
    out_ref[...] = jax.lax.dot_general(
        w2_ref[...], h, (((1,), (1,)), ((), ())),
        preferred_element_type=jnp.float32)


def _round_up(n, m):
    return ((n + m - 1) // m) * m


@jax.jit
def _forward(x, W, H, w1, b1, w2):
    B = x.shape[0]
    K = W.shape[1]

    user_idx = x[:, 0].astype(jnp.int32)
    item_idx = x[:, 1].astype(jnp.int32)

    eu = W.at[user_idx].get(mode="promise_in_bounds")
    ev = H.at[item_idx].get(mode="promise_in_bounds")

    TB = min(4096, _round_up(B, 256) // 2)
    TB = max(256, (TB // 256) * 256)
    B_pad = _round_up(B, TB)
    if B_pad != B:
        eu = jnp.pad(eu, ((0, B_pad - B), (0, 0)))
        ev = jnp.pad(ev, ((0, B_pad - B), (0, 0)))

    out = pl.pallas_call(
        _mlp_kernel,
        out_shape=jax.ShapeDtypeStruct((1, B_pad), jnp.float32),
        grid=(B_pad // TB,),
        in_specs=[
            pl.BlockSpec((TB, K), lambda i: (i, 0)),
            pl.BlockSpec((TB, K), lambda i: (i, 0)),
            pl.BlockSpec((K, K), lambda i: (0, 0)),
            pl.BlockSpec((K, K), lambda i: (0, 0)),
            pl.BlockSpec((1, K), lambda i: (0, 0)),
            pl.BlockSpec((1, K), lambda i: (0, 0)),
        ],
        out_specs=pl.BlockSpec((1, TB), lambda i: (0, i)),
        compiler_params=pltpu.CompilerParams(
            dimension_semantics=("parallel",),
        ),
    )(eu, ev, w1[:, :K].T, w1[:, K:].T, b1.reshape(1, K), w2.reshape(1, K))

    return out[0, :B].reshape(B, 1)


def kernel(x, W, H, w1, b1, w2):
    return _forward(x, W, H, w1, b1, w2)


# speedup vs baseline: 1.0215x; 1.0215x over previous
import jax
import jax.numpy as jnp
from jax.experimental import pallas as pl
from jax.experimental.pallas import tpu as pltpu


def _mlp_kernel(eu_ref, ev_ref, w1at_ref, w1bt_ref, b1_ref, w2_ref, out_ref):
    h = (jnp.dot(eu_ref[...], w1at_ref[...], preferred_element_type=jnp.float32)
         + jnp.dot(ev_ref[...], w1bt_ref[...], preferred_element_type=jnp.float32))
    h = jnp.maximum(h + b1_ref[...], 0.0)
    out_ref[...] = jax.lax.dot_general(
        w2_ref[...], h, (((1,), (1,)), ((), ())),
        preferred_element_type=jnp.float32)


def _round_up(n, m):
    return ((n + m - 1) // m) * m


@jax.jit
def _forward(x, W, H, w1, b1, w2):
    B = x.shape[0]
    K = W.shape[1]

    user_idx = x[:, 0].astype(jnp.int32)
    item_idx = x[:, 1].astype(jnp.int32)

    eu = W.at[user_idx].get(mode="promise_in_bounds")
    ev = H.at[item_idx].get(mode="promise_in_bounds")

    TB = min(16384, _round_up(B, 256) // 2)
    TB = max(256, (TB // 256) * 256)
    B_pad = _round_up(B, TB)
    if B_pad != B:
        eu = jnp.pad(eu, ((0, B_pad - B), (0, 0)))
        ev = jnp.pad(ev, ((0, B_pad - B), (0, 0)))

    out = pl.pallas_call(
        _mlp_kernel,
        out_shape=jax.ShapeDtypeStruct((1, B_pad), jnp.float32),
        grid=(B_pad // TB,),
        in_specs=[
            pl.BlockSpec((TB, K), lambda i: (i, 0)),
            pl.BlockSpec((TB, K), lambda i: (i, 0)),
            pl.BlockSpec((K, K), lambda i: (0, 0)),
            pl.BlockSpec((K, K), lambda i: (0, 0)),
            pl.BlockSpec((1, K), lambda i: (0, 0)),
            pl.BlockSpec((1, K), lambda i: (0, 0)),
        ],
        out_specs=pl.BlockSpec((1, TB), lambda i: (0, i)),
        compiler_params=pltpu.CompilerParams(
            dimension_semantics=("parallel",),
        ),
    )(eu, ev, w1[:, :K].T, w1[:, K:].T, b1.reshape(1, K), w2.reshape(1, K))

    return out[0, :B].reshape(B, 1)


def kernel(x, W, H, w1, b1, w2):
    return _forward(x, W, H, w1, b1, w2)


# speedup vs baseline: 1.0263x; 1.0047x over previous
import jax
import jax.numpy as jnp
from jax.experimental import pallas as pl
from jax.experimental.pallas import tpu as pltpu


def _mlp_kernel(eu_ref, ev_ref, w1a_ref, w1b_ref, b1_ref, w2_ref, out_ref):
    dims = (((1,), (1,)), ((), ()))
    h = (jax.lax.dot_general(w1a_ref[...], eu_ref[...], dims,
                             preferred_element_type=jnp.float32)
         + jax.lax.dot_general(w1b_ref[...], ev_ref[...], dims,
                               preferred_element_type=jnp.float32))
    h = jnp.maximum(h + b1_ref[...], 0.0)
    out_ref[...] = jnp.sum(w2_ref[...] * h, axis=0, keepdims=True)


def _round_up(n, m):
    return ((n + m - 1) // m) * m


@jax.jit
def _forward(x, W, H, w1, b1, w2):
    B = x.shape[0]
    K = W.shape[1]

    user_idx = x[:, 0].astype(jnp.int32)
    item_idx = x[:, 1].astype(jnp.int32)

    eu = W.at[user_idx].get(mode="promise_in_bounds")
    ev = H.at[item_idx].get(mode="promise_in_bounds")

    TB = min(16384, _round_up(B, 256) // 2)
    TB = max(256, (TB // 256) * 256)
    B_pad = _round_up(B, TB)
    if B_pad != B:
        eu = jnp.pad(eu, ((0, B_pad - B), (0, 0)))
        ev = jnp.pad(ev, ((0, B_pad - B), (0, 0)))

    out = pl.pallas_call(
        _mlp_kernel,
        out_shape=jax.ShapeDtypeStruct((1, B_pad), jnp.float32),
        grid=(B_pad // TB,),
        in_specs=[
            pl.BlockSpec((TB, K), lambda i: (i, 0)),
            pl.BlockSpec((TB, K), lambda i: (i, 0)),
            pl.BlockSpec((K, K), lambda i: (0, 0)),
            pl.BlockSpec((K, K), lambda i: (0, 0)),
            pl.BlockSpec((K, 1), lambda i: (0, 0)),
            pl.BlockSpec((K, 1), lambda i: (0, 0)),
        ],
        out_specs=pl.BlockSpec((1, TB), lambda i: (0, i)),
        compiler_params=pltpu.CompilerParams(
            dimension_semantics=("parallel",),
        ),
    )(eu, ev, w1[:, :K], w1[:, K:], b1.reshape(K, 1), w2.reshape(K, 1))

    return out[0, :B].reshape(B, 1)


def kernel(x, W, H, w1, b1, w2):
    return _forward(x, W, H, w1, b1, w2)
